# baseline (speedup 1.0000x reference)
import functools

import jax
import jax.numpy as jnp
from jax import lax
from jax.experimental import pallas as pl
from jax.experimental.pallas import tpu as pltpu
from jax.experimental.pallas import tpu_sc as plsc

NUM_EMBEDDINGS = 1000000
D = 32
B = 16384
H = 50
TOT = B * H
NC = 2
NS = 16
NW = NC * NS
PER_W = TOT // NW
CHUNK = 1024
NG = PER_W // CHUNK
NBUF = 2


def _emb_body(x_hbm, table_hbm, out_hbm, idx_all, in0, in1, f0, f1,
              g0, g1, w0, w1):
    wid = lax.axis_index("s") * NC + lax.axis_index("c")
    base = wid * PER_W
    pltpu.sync_copy(x_hbm.at[pl.ds(base, PER_W)], idx_all)
    ins = (in0, in1)
    outs = (f0, f1)
    gsem = (g0, g1)
    wsem = (w0, w1)

    def gather(g):
        b = g % NBUF
        return pltpu.async_copy(
            table_hbm.at[idx_all.at[pl.ds(g * CHUNK, CHUNK)]], ins[b], gsem[b])

    col_even = 2 * lax.iota(jnp.int32, 16)
    col_odd = col_even + 1

    def expand(b):
        src = ins[b]
        dst = outs[b]

        def body(r, carry):
            w = plsc.bitcast(src[r, :], jnp.int32)
            lo = plsc.bitcast(lax.shift_left(w, 16), jnp.float32)
            hi = plsc.bitcast(lax.bitwise_and(w, jnp.int32(-65536)),
                              jnp.float32)
            rows = jnp.zeros((16,), jnp.int32) + r
            plsc.store_scatter(dst, [rows, col_even], lo)
            plsc.store_scatter(dst, [rows, col_odd], hi)
            return carry

        lax.fori_loop(0, CHUNK, body, 0)

    gh = [None] * NG
    wh = [None] * NG
    for g in range(NBUF):
        gh[g] = gather(g)
    for g in range(NG):
        b = g % NBUF
        gh[g].wait()
        expand(b)
        wh[g] = pltpu.async_copy(
            outs[b], out_hbm.at[pl.ds(base + g * CHUNK, CHUNK)], wsem[b])
        if g + NBUF < NG:
            wh[g].wait()
            gh[g + NBUF] = gather(g + NBUF)
    for g in range(NG - NBUF, NG):
        wh[g].wait()


_emb = functools.partial(
    pl.kernel,
    mesh=plsc.VectorSubcoreMesh(core_axis_name="c", subcore_axis_name="s"),
    out_type=jax.ShapeDtypeStruct((TOT, D), jnp.float32),
    scratch_types=[
        pltpu.VMEM((PER_W,), jnp.int32),
        pltpu.VMEM((CHUNK, D), jnp.bfloat16),
        pltpu.VMEM((CHUNK, D), jnp.bfloat16),
        pltpu.VMEM((CHUNK, D), jnp.float32),
        pltpu.VMEM((CHUNK, D), jnp.float32),
        pltpu.SemaphoreType.DMA,
        pltpu.SemaphoreType.DMA,
        pltpu.SemaphoreType.DMA,
        pltpu.SemaphoreType.DMA,
    ],
    compiler_params=pltpu.CompilerParams(use_tc_tiling_on_sc=False,
                                         needs_layout_passes=False),
)(_emb_body)


def kernel(x, table):
    t16 = table.astype(jnp.bfloat16)
    out = _emb(x.reshape(TOT), t16)
    return out.reshape(B, H, D)


# speedup vs baseline: 1.0288x; 1.0288x over previous
import functools

import jax
import jax.numpy as jnp
from jax import lax
from jax.experimental import pallas as pl
from jax.experimental.pallas import tpu as pltpu
from jax.experimental.pallas import tpu_sc as plsc

NUM_EMBEDDINGS = 1000000
NROWS = NUM_EMBEDDINGS + 1
D = 32
DW = 16
B = 16384
H = 50
TOT = B * H
NC = 2
NS = 16
NW = NC * NS
ROUND = jnp.int32(0x8000)
HIMASK = jnp.int32(-65536)

ROWS_P = NUM_EMBEDDINGS // NW
PCH = 1250
NPC = ROWS_P // PCH


def _pack_rows(src, dst, n, unroll):
    def body(r, carry):
        a = plsc.bitcast(src[r, pl.ds(0, 16)], jnp.int32)
        b = plsc.bitcast(src[r, pl.ds(16, 16)], jnp.int32)
        w = lax.bitwise_or(
            lax.shift_right_logical(a + ROUND, 16),
            lax.bitwise_and(b + ROUND, HIMASK))
        dst[r, :] = w
        return carry

    lax.fori_loop(0, n, body, 0, unroll=unroll)


def _pack_body(tbl_hbm, packed_hbm, in0, in1, pk0, pk1, i0, i1, o0, o1):
    wid = lax.axis_index("s") * NC + lax.axis_index("c")
    base = wid * ROWS_P
    ins = (in0, in1)
    pks = (pk0, pk1)
    isem = (i0, i1)
    osem = (o0, o1)

    def load(c):
        b = c % 2
        return pltpu.async_copy(
            tbl_hbm.at[pl.ds(base + c * PCH, PCH)], ins[b], isem[b])

    ih = [None] * NPC
    oh = [None] * NPC
    for c in range(2):
        ih[c] = load(c)
    for c in range(NPC):
        b = c % 2
        ih[c].wait()
        if c >= 2:
            oh[c - 2].wait()
        _pack_rows(ins[b], pks[b], PCH, 5)
        oh[c] = pltpu.async_copy(
            pks[b], packed_hbm.at[pl.ds(base + c * PCH, PCH)], osem[b])
        if c + 2 < NPC:
            ih[c + 2] = load(c + 2)
    oh[NPC - 2].wait()
    oh[NPC - 1].wait()
    pltpu.sync_copy(tbl_hbm.at[pl.ds(NUM_EMBEDDINGS, 1)], ins[0].at[pl.ds(0, 1)])
    _pack_rows(ins[0], pks[0], 1, 1)
    pltpu.sync_copy(pks[0].at[pl.ds(0, 1)],
                    packed_hbm.at[pl.ds(NUM_EMBEDDINGS, 1)])


_pack = functools.partial(
    pl.kernel,
    mesh=plsc.VectorSubcoreMesh(core_axis_name="c", subcore_axis_name="s"),
    out_type=jax.ShapeDtypeStruct((NROWS, DW), jnp.int32),
    scratch_types=[
        pltpu.VMEM((PCH, D), jnp.float32),
        pltpu.VMEM((PCH, D), jnp.float32),
        pltpu.VMEM((PCH, DW), jnp.int32),
        pltpu.VMEM((PCH, DW), jnp.int32),
        pltpu.SemaphoreType.DMA,
        pltpu.SemaphoreType.DMA,
        pltpu.SemaphoreType.DMA,
        pltpu.SemaphoreType.DMA,
    ],
    compiler_params=pltpu.CompilerParams(use_tc_tiling_on_sc=False,
                                         needs_layout_passes=False),
)(_pack_body)

PER_W = TOT // NW
CHUNK = 1024
NG = PER_W // CHUNK
NBUF = 2


def _emb_body(x_hbm, packed_hbm, out_hbm, idx_all, in0, in1, f0, f1,
              g0, g1, w0, w1):
    wid = lax.axis_index("s") * NC + lax.axis_index("c")
    base = wid * PER_W
    pltpu.sync_copy(x_hbm.at[pl.ds(base, PER_W)], idx_all)
    ins = (in0, in1)
    outs = (f0, f1)
    gsem = (g0, g1)
    wsem = (w0, w1)

    def gather(g):
        b = g % NBUF
        return pltpu.async_copy(
            packed_hbm.at[idx_all.at[pl.ds(g * CHUNK, CHUNK)]], ins[b], gsem[b])

    def expand(b):
        src = ins[b]
        dst = outs[b]

        def body(r, carry):
            w = src[r, :]
            lo = plsc.bitcast(lax.shift_left(w, 16), jnp.float32)
            hi = plsc.bitcast(lax.bitwise_and(w, HIMASK), jnp.float32)
            dst[r, pl.ds(0, 16)] = lo
            dst[r, pl.ds(16, 16)] = hi
            return carry

        lax.fori_loop(0, CHUNK, body, 0, unroll=4)

    gh = [None] * NG
    wh = [None] * NG
    for g in range(NBUF):
        gh[g] = gather(g)
    for g in range(NG):
        b = g % NBUF
        gh[g].wait()
        expand(b)
        wh[g] = pltpu.async_copy(
            outs[b], out_hbm.at[pl.ds(base + g * CHUNK, CHUNK)], wsem[b])
        if g + NBUF < NG:
            wh[g].wait()
            gh[g + NBUF] = gather(g + NBUF)
    for g in range(NG - NBUF, NG):
        wh[g].wait()


_emb = functools.partial(
    pl.kernel,
    mesh=plsc.VectorSubcoreMesh(core_axis_name="c", subcore_axis_name="s"),
    out_type=jax.ShapeDtypeStruct((TOT, D), jnp.float32),
    scratch_types=[
        pltpu.VMEM((PER_W,), jnp.int32),
        pltpu.VMEM((CHUNK, DW), jnp.int32),
        pltpu.VMEM((CHUNK, DW), jnp.int32),
        pltpu.VMEM((CHUNK, D), jnp.float32),
        pltpu.VMEM((CHUNK, D), jnp.float32),
        pltpu.SemaphoreType.DMA,
        pltpu.SemaphoreType.DMA,
        pltpu.SemaphoreType.DMA,
        pltpu.SemaphoreType.DMA,
    ],
    compiler_params=pltpu.CompilerParams(use_tc_tiling_on_sc=False,
                                         needs_layout_passes=False),
)(_emb_body)


def kernel(x, table):
    packed = _pack(table)
    out = _emb(x.reshape(TOT), packed)
    return out.reshape(B, H, D)


# speedup vs baseline: 1.1968x; 1.1633x over previous
import functools

import jax
import jax.numpy as jnp
from jax import lax
from jax.experimental import pallas as pl
from jax.experimental.pallas import tpu as pltpu
from jax.experimental.pallas import tpu_sc as plsc

NUM_EMBEDDINGS = 1000000
D = 32
B = 16384
H = 50
TOT = B * H
NC = 2
NS = 16
NW = NC * NS
PER_W = TOT // NW
CHUNK = 1024
NG = PER_W // CHUNK
NBUF = 3


def _emb_body(x_hbm, table_hbm, out_hbm, idx_all, rows0, rows1, rows2,
              g0, g1, g2, w0, w1, w2):
    wid = lax.axis_index("s") * NC + lax.axis_index("c")
    base = wid * PER_W
    pltpu.sync_copy(x_hbm.at[pl.ds(base, PER_W)], idx_all)
    rows = (rows0, rows1, rows2)
    gsem = (g0, g1, g2)
    wsem = (w0, w1, w2)

    def gather(g):
        b = g % NBUF
        return pltpu.async_copy(
            table_hbm.at[idx_all.at[pl.ds(g * CHUNK, CHUNK)]], rows[b], gsem[b])

    gh = [None] * NG
    wh = [None] * NG
    for g in range(NBUF):
        gh[g] = gather(g)
    for g in range(NG):
        b = g % NBUF
        gh[g].wait()
        wh[g] = pltpu.async_copy(
            rows[b], out_hbm.at[pl.ds(base + g * CHUNK, CHUNK)], wsem[b])
        if g + NBUF < NG:
            wh[g].wait()
            gh[g + NBUF] = gather(g + NBUF)
    for g in range(NG - NBUF, NG):
        wh[g].wait()


_emb = functools.partial(
    pl.kernel,
    mesh=plsc.VectorSubcoreMesh(core_axis_name="c", subcore_axis_name="s"),
    out_type=jax.ShapeDtypeStruct((TOT, D), jnp.float32),
    scratch_types=[
        pltpu.VMEM((PER_W,), jnp.int32),
        pltpu.VMEM((CHUNK, D), jnp.float32),
        pltpu.VMEM((CHUNK, D), jnp.float32),
        pltpu.VMEM((CHUNK, D), jnp.float32),
        pltpu.SemaphoreType.DMA,
        pltpu.SemaphoreType.DMA,
        pltpu.SemaphoreType.DMA,
        pltpu.SemaphoreType.DMA,
        pltpu.SemaphoreType.DMA,
        pltpu.SemaphoreType.DMA,
    ],
    compiler_params=pltpu.CompilerParams(use_tc_tiling_on_sc=False),
)(_emb_body)


def kernel(x, table):
    out = _emb(x.reshape(TOT), table)
    return out.reshape(B, H, D)
